# baseline (speedup 1.0000x reference)
import functools

import jax
import jax.numpy as jnp
from jax import lax
from jax.experimental import pallas as pl
from jax.experimental.pallas import tpu as pltpu
from jax.experimental.pallas import tpu_sc as plsc

N = 10000
E = 320000
H = 128
G = 64

NC = 2
NS = 16
NW = NC * NS
E_PER_W = E // NW
CH = 128
CPW = 80
PADE = CPW * CH - E_PER_W
NPAIR = CPW // 2
N_PAD = 10240
ROWS_PER_TILE = N_PAD // NS
DW = 16

_mesh = plsc.VectorSubcoreMesh(core_axis_name="c", subcore_axis_name="s")


@functools.partial(
    pl.kernel,
    out_type=jax.ShapeDtypeStruct((NC, N_PAD, DW), jnp.float32),
    mesh=_mesh,
    scratch_types=[
        pltpu.VMEM((CPW, CH), jnp.int32),
        pltpu.VMEM((CH, DW), jnp.float32),
        pltpu.VMEM_SHARED((N_PAD, DW), jnp.float32),
        pltpu.SemaphoreType.DMA,
    ],
)
def _deg_kernel(dstp_hbm, out_hbm, idx_v, ones_v, acc, sem):
    cid = lax.axis_index("c")
    sid = lax.axis_index("s")
    w = sid * NC + cid
    pltpu.sync_copy(dstp_hbm.at[w], idx_v)
    z16 = jnp.zeros((16,), jnp.float32)

    def zrow(i, _):
        ones_v[i, :] = z16
        return ()

    lax.fori_loop(0, CH, zrow, ())
    r0 = sid * ROWS_PER_TILE
    for k in range(ROWS_PER_TILE // CH):
        pltpu.sync_copy(ones_v, acc.at[pl.ds(r0 + k * CH, CH)])

    one16 = jnp.ones((16,), jnp.float32)

    def orow(i, _):
        ones_v[i, :] = one16
        return ()

    lax.fori_loop(0, CH, orow, ())
    plsc.subcore_barrier()

    def grp(i, _):
        pltpu.sync_copy(ones_v, acc.at[idx_v.at[i]], add=True)
        return ()

    lax.fori_loop(0, CPW, grp, ())

    plsc.subcore_barrier()
    pltpu.sync_copy(acc.at[pl.ds(r0, ROWS_PER_TILE)],
                    out_hbm.at[cid, pl.ds(r0, ROWS_PER_TILE)])


@functools.partial(
    pl.kernel,
    out_type=jax.ShapeDtypeStruct((NC, N_PAD, H), jnp.float32),
    mesh=_mesh,
    scratch_types=[
        pltpu.VMEM((CH,), jnp.int32),
        pltpu.VMEM((CH,), jnp.int32),
        pltpu.VMEM((CH,), jnp.int32),
        pltpu.VMEM((CH,), jnp.int32),
        pltpu.VMEM((CH, H), jnp.float32),
        pltpu.VMEM((CH, H), jnp.float32),
        pltpu.VMEM_SHARED((N_PAD, H), jnp.float32),
        pltpu.SemaphoreType.DMA,
        pltpu.SemaphoreType.DMA,
        pltpu.SemaphoreType.DMA,
        pltpu.SemaphoreType.DMA,
    ],
)
def _agg_kernel(srcp_hbm, dstp_hbm, y_hbm, out_hbm,
                src_a, src_b, dst_a, dst_b, rows_a, rows_b, acc,
                sem_ia, sem_ib, sem_ga, sem_gb):
    cid = lax.axis_index("c")
    sid = lax.axis_index("s")
    w = sid * NC + cid
    base_w = w * (CPW * CH)
    z16 = jnp.zeros((16,), jnp.float32)

    def zrow(i, _):
        for t in range(H // 16):
            rows_a[i, pl.ds(t * 16, 16)] = z16
        return ()

    lax.fori_loop(0, CH, zrow, ())
    r0 = sid * ROWS_PER_TILE
    for k in range(ROWS_PER_TILE // CH):
        pltpu.sync_copy(rows_a, acc.at[pl.ds(r0 + k * CH, CH)])
    plsc.subcore_barrier()

    def chunk(j, _):
        base = base_w + j * CH
        pltpu.sync_copy(srcp_hbm.at[pl.ds(base, CH)], src_a)
        pltpu.sync_copy(dstp_hbm.at[pl.ds(base, CH)], dst_a)
        pltpu.async_copy(y_hbm.at[src_a], rows_a, sem_ga).wait()
        pltpu.sync_copy(rows_a, acc.at[dst_a], add=True)
        return ()

    lax.fori_loop(0, CPW, chunk, ())

    plsc.subcore_barrier()
    pltpu.sync_copy(acc.at[pl.ds(r0, ROWS_PER_TILE)],
                    out_hbm.at[cid, pl.ds(r0, ROWS_PER_TILE)])


_BLK = 1000
_NBLK = N // _BLK


def _pre1_body(degp_ref, x_ref, w_ref, y_ref, dinv_ref):
    deg = jnp.sum(degp_ref[...], axis=1) * (1.0 / DW) + 1.0
    dinv = lax.rsqrt(deg)
    xw = jnp.dot(x_ref[...], w_ref[...], preferred_element_type=jnp.float32)
    y_ref[...] = dinv[:, None] * xw
    dinv_ref[...] = dinv[:, None]


_pre1 = pl.pallas_call(
    _pre1_body,
    grid=(_NBLK,),
    in_specs=[
        pl.BlockSpec((_BLK, NC * DW), lambda i: (i, 0)),
        pl.BlockSpec((_BLK, H), lambda i: (i, 0)),
        pl.BlockSpec((H, H), lambda i: (0, 0)),
    ],
    out_specs=(
        pl.BlockSpec((_BLK, H), lambda i: (i, 0)),
        pl.BlockSpec((_BLK, 1), lambda i: (i, 0)),
    ),
    out_shape=(
        jax.ShapeDtypeStruct((N, H), jnp.float32),
        jax.ShapeDtypeStruct((N, 1), jnp.float32),
    ),
)


def _trans_body(aggp_ref, y_ref, dinv_ref, b_ref, w_ref, out_ref):
    a = aggp_ref[0] + aggp_ref[1] + y_ref[...]
    h = jnp.tanh(dinv_ref[...] * a + b_ref[...])
    out_ref[...] = dinv_ref[...] * jnp.dot(
        h, w_ref[...], preferred_element_type=jnp.float32)


_trans = pl.pallas_call(
    _trans_body,
    grid=(_NBLK,),
    in_specs=[
        pl.BlockSpec((NC, _BLK, H), lambda i: (0, i, 0)),
        pl.BlockSpec((_BLK, H), lambda i: (i, 0)),
        pl.BlockSpec((_BLK, 1), lambda i: (i, 0)),
        pl.BlockSpec((1, H), lambda i: (0, 0)),
        pl.BlockSpec((H, H), lambda i: (0, 0)),
    ],
    out_specs=pl.BlockSpec((_BLK, H), lambda i: (i, 0)),
    out_shape=jax.ShapeDtypeStruct((N, H), jnp.float32),
)


def _final_body(aggp_ref, y_ref, dinv_ref, b_ref, batch_ref, wl_ref, bl_ref,
                out_ref, seg_ref, cnt_ref):
    i = pl.program_id(0)

    @pl.when(i == 0)
    def _():
        seg_ref[...] = jnp.zeros_like(seg_ref)
        cnt_ref[...] = jnp.zeros_like(cnt_ref)

    a = aggp_ref[0] + aggp_ref[1] + y_ref[...]
    h = dinv_ref[...] * a + b_ref[...]
    bt = batch_ref[...].reshape(1, _BLK)
    oh = (lax.broadcasted_iota(jnp.int32, (G, _BLK), 0)
          == jnp.broadcast_to(bt, (G, _BLK))).astype(jnp.float32)
    seg_ref[...] += jnp.dot(oh, h, preferred_element_type=jnp.float32)
    cnt_ref[...] += jnp.sum(oh, axis=1, keepdims=True)

    @pl.when(i == _NBLK - 1)
    def _():
        cnt = cnt_ref[...]
        mean = jnp.where(cnt > 0, seg_ref[...] / jnp.maximum(cnt, 1.0), 0.0)
        out_ref[...] = jnp.dot(
            mean, wl_ref[...], preferred_element_type=jnp.float32) + bl_ref[...]


_final = pl.pallas_call(
    _final_body,
    grid=(_NBLK,),
    in_specs=[
        pl.BlockSpec((NC, _BLK, H), lambda i: (0, i, 0)),
        pl.BlockSpec((_BLK, H), lambda i: (i, 0)),
        pl.BlockSpec((_BLK, 1), lambda i: (i, 0)),
        pl.BlockSpec((1, H), lambda i: (0, 0)),
        pl.BlockSpec((1, 1, _BLK), lambda i: (i, 0, 0)),
        pl.BlockSpec((H, 1), lambda i: (0, 0)),
        pl.BlockSpec((1, 1), lambda i: (0, 0)),
    ],
    out_specs=pl.BlockSpec((G, 1), lambda i: (0, 0)),
    out_shape=jax.ShapeDtypeStruct((G, 1), jnp.float32),
    scratch_shapes=[
        pltpu.VMEM((G, H), jnp.float32),
        pltpu.VMEM((G, 1), jnp.float32),
    ],
)


def kernel(x, edge_index, batch, W1, b1, W2, b2, W3, b3, Wl, bl):
    src = edge_index[0]
    dst = edge_index[1]
    srcp = jnp.concatenate(
        [src.reshape(NW, E_PER_W),
         jnp.zeros((NW, PADE), jnp.int32)], axis=1).reshape(NW, CPW, CH)
    dstp = jnp.concatenate(
        [dst.reshape(NW, E_PER_W),
         jnp.full((NW, PADE), N_PAD - 1, jnp.int32)], axis=1).reshape(
             NW, CPW, CH)
    srcp_f = srcp.reshape(-1)
    dstp_f = dstp.reshape(-1)
    deg_p = _deg_kernel(dstp)
    y1, dinv = _pre1(deg_p.transpose(1, 0, 2).reshape(N_PAD, NC * DW), x, W1)
    agg1 = _agg_kernel(srcp_f, dstp_f, y1)
    y2 = _trans(agg1, y1, dinv, b1.reshape(1, H), W2)
    agg2 = _agg_kernel(srcp_f, dstp_f, y2)
    y3 = _trans(agg2, y2, dinv, b2.reshape(1, H), W3)
    agg3 = _agg_kernel(srcp_f, dstp_f, y3)
    out = _final(agg3, y3, dinv, b3.reshape(1, H),
                 batch.reshape(_NBLK, 1, _BLK), Wl, bl.reshape(1, 1))
    return out


# speedup vs baseline: 1.9314x; 1.9314x over previous
import functools

import jax
import jax.numpy as jnp
from jax import lax
from jax.experimental import pallas as pl
from jax.experimental.pallas import tpu as pltpu
from jax.experimental.pallas import tpu_sc as plsc

N = 10000
E = 320000
H = 128
G = 64

NC = 2
NS = 16
NW = NC * NS
E_PER_W = E // NW
CH = 128
N_FULL = E_PER_W // CH
TAIL = E_PER_W - N_FULL * CH
N_PAD = 10240
ROWS_PER_TILE = N_PAD // NS

_mesh = plsc.VectorSubcoreMesh(core_axis_name="c", subcore_axis_name="s")


DW = 16


@functools.partial(
    pl.kernel,
    out_type=jax.ShapeDtypeStruct((NC, N_PAD, DW), jnp.float32),
    mesh=_mesh,
    scratch_types=[
        pltpu.VMEM((CH,), jnp.int32),
        pltpu.VMEM((CH, DW), jnp.float32),
        pltpu.VMEM((TAIL,), jnp.int32),
        pltpu.VMEM_SHARED((N_PAD, DW), jnp.float32),
    ],
)
def _deg_kernel(dst_hbm, out_hbm, dst_v, ones_v, dst_t, acc):
    cid = lax.axis_index("c")
    sid = lax.axis_index("s")
    w = sid * NC + cid
    z16 = jnp.zeros((16,), jnp.float32)

    def zrow(i, _):
        ones_v[i, :] = z16
        return ()

    lax.fori_loop(0, CH, zrow, ())
    r0 = sid * ROWS_PER_TILE
    for k in range(ROWS_PER_TILE // CH):
        pltpu.sync_copy(ones_v, acc.at[pl.ds(r0 + k * CH, CH)])

    one16 = jnp.ones((16,), jnp.float32)

    def orow(i, _):
        ones_v[i, :] = one16
        return ()

    lax.fori_loop(0, CH, orow, ())
    plsc.subcore_barrier()

    base_w = w * E_PER_W

    def chunk(j, _):
        base = base_w + j * CH
        pltpu.sync_copy(dst_hbm.at[pl.ds(base, CH)], dst_v)
        pltpu.sync_copy(ones_v, acc.at[dst_v], add=True)
        return ()

    lax.fori_loop(0, N_FULL, chunk, ())

    tbase = base_w + N_FULL * CH
    pltpu.sync_copy(dst_hbm.at[pl.ds(tbase, TAIL)], dst_t)
    pltpu.sync_copy(ones_v.at[pl.ds(0, TAIL)], acc.at[dst_t], add=True)

    plsc.subcore_barrier()
    pltpu.sync_copy(acc.at[pl.ds(r0, ROWS_PER_TILE)],
                    out_hbm.at[cid, pl.ds(r0, ROWS_PER_TILE)])


@functools.partial(
    pl.kernel,
    out_type=jax.ShapeDtypeStruct((NC, N_PAD, H), jnp.float32),
    mesh=_mesh,
    scratch_types=[
        pltpu.VMEM((CH,), jnp.int32),
        pltpu.VMEM((CH,), jnp.int32),
        pltpu.VMEM((CH, H), jnp.float32),
        pltpu.VMEM((TAIL,), jnp.int32),
        pltpu.VMEM((TAIL,), jnp.int32),
        pltpu.VMEM((TAIL, H), jnp.float32),
        pltpu.VMEM_SHARED((N_PAD, H), jnp.float32),
        pltpu.SemaphoreType.DMA,
    ],
)
def _agg_kernel(src_hbm, dst_hbm, y_hbm, out_hbm,
                src_v, dst_v, rows_v, src_t, dst_t, rows_t, acc, sem):
    cid = lax.axis_index("c")
    sid = lax.axis_index("s")
    w = sid * NC + cid
    z16 = jnp.zeros((16,), jnp.float32)

    def zrow(i, _):
        for t in range(H // 16):
            rows_v[i, pl.ds(t * 16, 16)] = z16
        return ()

    lax.fori_loop(0, CH, zrow, ())
    r0 = sid * ROWS_PER_TILE
    for k in range(ROWS_PER_TILE // CH):
        pltpu.sync_copy(rows_v, acc.at[pl.ds(r0 + k * CH, CH)])
    plsc.subcore_barrier()

    base_w = w * E_PER_W

    def chunk(j, _):
        base = base_w + j * CH
        pltpu.sync_copy(src_hbm.at[pl.ds(base, CH)], src_v)
        pltpu.sync_copy(dst_hbm.at[pl.ds(base, CH)], dst_v)
        pltpu.async_copy(y_hbm.at[src_v], rows_v, sem).wait()
        pltpu.sync_copy(rows_v, acc.at[dst_v], add=True)
        return ()

    lax.fori_loop(0, N_FULL, chunk, ())

    tbase = base_w + N_FULL * CH
    pltpu.sync_copy(src_hbm.at[pl.ds(tbase, TAIL)], src_t)
    pltpu.sync_copy(dst_hbm.at[pl.ds(tbase, TAIL)], dst_t)
    pltpu.async_copy(y_hbm.at[src_t], rows_t, sem).wait()
    pltpu.sync_copy(rows_t, acc.at[dst_t], add=True)

    plsc.subcore_barrier()
    pltpu.sync_copy(acc.at[pl.ds(r0, ROWS_PER_TILE)],
                    out_hbm.at[cid, pl.ds(r0, ROWS_PER_TILE)])


_BLK = 1000
_NBLK = N // _BLK


def _pre1_body(degp_ref, x_ref, w_ref, y_ref, dinv_ref):
    deg = jnp.sum(degp_ref[...], axis=1) * (1.0 / DW) + 1.0
    dinv = lax.rsqrt(deg)
    xw = jnp.dot(x_ref[...], w_ref[...], preferred_element_type=jnp.float32)
    y_ref[...] = dinv[:, None] * xw
    dinv_ref[...] = dinv[:, None]


_pre1 = pl.pallas_call(
    _pre1_body,
    grid=(_NBLK,),
    in_specs=[
        pl.BlockSpec((_BLK, NC * DW), lambda i: (i, 0)),
        pl.BlockSpec((_BLK, H), lambda i: (i, 0)),
        pl.BlockSpec((H, H), lambda i: (0, 0)),
    ],
    out_specs=(
        pl.BlockSpec((_BLK, H), lambda i: (i, 0)),
        pl.BlockSpec((_BLK, 1), lambda i: (i, 0)),
    ),
    out_shape=(
        jax.ShapeDtypeStruct((N, H), jnp.float32),
        jax.ShapeDtypeStruct((N, 1), jnp.float32),
    ),
)


def _trans_body(aggp_ref, y_ref, dinv_ref, b_ref, w_ref, out_ref):
    a = aggp_ref[0] + aggp_ref[1] + y_ref[...]
    h = jnp.tanh(dinv_ref[...] * a + b_ref[...])
    out_ref[...] = dinv_ref[...] * jnp.dot(
        h, w_ref[...], preferred_element_type=jnp.float32)


_trans = pl.pallas_call(
    _trans_body,
    grid=(_NBLK,),
    in_specs=[
        pl.BlockSpec((NC, _BLK, H), lambda i: (0, i, 0)),
        pl.BlockSpec((_BLK, H), lambda i: (i, 0)),
        pl.BlockSpec((_BLK, 1), lambda i: (i, 0)),
        pl.BlockSpec((1, H), lambda i: (0, 0)),
        pl.BlockSpec((H, H), lambda i: (0, 0)),
    ],
    out_specs=pl.BlockSpec((_BLK, H), lambda i: (i, 0)),
    out_shape=jax.ShapeDtypeStruct((N, H), jnp.float32),
)


def _final_body(aggp_ref, y_ref, dinv_ref, b_ref, batch_ref, wl_ref, bl_ref,
                out_ref, seg_ref, cnt_ref):
    i = pl.program_id(0)

    @pl.when(i == 0)
    def _():
        seg_ref[...] = jnp.zeros_like(seg_ref)
        cnt_ref[...] = jnp.zeros_like(cnt_ref)

    a = aggp_ref[0] + aggp_ref[1] + y_ref[...]
    h = dinv_ref[...] * a + b_ref[...]
    bt = batch_ref[...].reshape(1, _BLK)
    oh = (lax.broadcasted_iota(jnp.int32, (G, _BLK), 0)
          == jnp.broadcast_to(bt, (G, _BLK))).astype(jnp.float32)
    seg_ref[...] += jnp.dot(oh, h, preferred_element_type=jnp.float32)
    cnt_ref[...] += jnp.sum(oh, axis=1, keepdims=True)

    @pl.when(i == _NBLK - 1)
    def _():
        cnt = cnt_ref[...]
        mean = jnp.where(cnt > 0, seg_ref[...] / jnp.maximum(cnt, 1.0), 0.0)
        out_ref[...] = jnp.dot(
            mean, wl_ref[...], preferred_element_type=jnp.float32) + bl_ref[...]


_final = pl.pallas_call(
    _final_body,
    grid=(_NBLK,),
    in_specs=[
        pl.BlockSpec((NC, _BLK, H), lambda i: (0, i, 0)),
        pl.BlockSpec((_BLK, H), lambda i: (i, 0)),
        pl.BlockSpec((_BLK, 1), lambda i: (i, 0)),
        pl.BlockSpec((1, H), lambda i: (0, 0)),
        pl.BlockSpec((1, 1, _BLK), lambda i: (i, 0, 0)),
        pl.BlockSpec((H, 1), lambda i: (0, 0)),
        pl.BlockSpec((1, 1), lambda i: (0, 0)),
    ],
    out_specs=pl.BlockSpec((G, 1), lambda i: (0, 0)),
    out_shape=jax.ShapeDtypeStruct((G, 1), jnp.float32),
    scratch_shapes=[
        pltpu.VMEM((G, H), jnp.float32),
        pltpu.VMEM((G, 1), jnp.float32),
    ],
)


def kernel(x, edge_index, batch, W1, b1, W2, b2, W3, b3, Wl, bl):
    src = edge_index[0]
    dst = edge_index[1]
    deg_p = _deg_kernel(dst)
    y1, dinv = _pre1(deg_p.transpose(1, 0, 2).reshape(N_PAD, NC * DW), x, W1)
    agg1 = _agg_kernel(src, dst, y1)
    y2 = _trans(agg1, y1, dinv, b1.reshape(1, H), W2)
    agg2 = _agg_kernel(src, dst, y2)
    y3 = _trans(agg2, y2, dinv, b2.reshape(1, H), W3)
    agg3 = _agg_kernel(src, dst, y3)
    out = _final(agg3, y3, dinv, b3.reshape(1, H),
                 batch.reshape(_NBLK, 1, _BLK), Wl, bl.reshape(1, 1))
    return out


# speedup vs baseline: 2.8472x; 1.4741x over previous
import functools

import jax
import jax.numpy as jnp
from jax import lax
from jax.experimental import pallas as pl
from jax.experimental.pallas import tpu as pltpu
from jax.experimental.pallas import tpu_sc as plsc

N = 10000
E = 320000
H = 128
G = 64

NC = 2
NS = 16
NW = NC * NS
E_PER_W = E // NW
CH = 128
N_FULL = E_PER_W // CH
TAIL = E_PER_W - N_FULL * CH
N_PAD = 10240
ROWS_PER_TILE = N_PAD // NS

_mesh = plsc.VectorSubcoreMesh(core_axis_name="c", subcore_axis_name="s")


DW = 16


@functools.partial(
    pl.kernel,
    out_type=jax.ShapeDtypeStruct((NC, N_PAD, DW), jnp.float32),
    mesh=_mesh,
    scratch_types=[
        pltpu.VMEM((CH,), jnp.int32),
        pltpu.VMEM((CH, DW), jnp.float32),
        pltpu.VMEM((TAIL,), jnp.int32),
        pltpu.VMEM_SHARED((N_PAD, DW), jnp.float32),
    ],
)
def _deg_kernel(dst_hbm, out_hbm, dst_v, ones_v, dst_t, acc):
    cid = lax.axis_index("c")
    sid = lax.axis_index("s")
    w = sid * NC + cid
    z16 = jnp.zeros((16,), jnp.float32)

    def zrow(i, _):
        ones_v[i, :] = z16
        return ()

    lax.fori_loop(0, CH, zrow, ())
    r0 = sid * ROWS_PER_TILE
    for k in range(ROWS_PER_TILE // CH):
        pltpu.sync_copy(ones_v, acc.at[pl.ds(r0 + k * CH, CH)])

    one16 = jnp.ones((16,), jnp.float32)

    def orow(i, _):
        ones_v[i, :] = one16
        return ()

    lax.fori_loop(0, CH, orow, ())
    plsc.subcore_barrier()

    base_w = w * E_PER_W

    def chunk(j, _):
        base = base_w + j * CH
        pltpu.sync_copy(dst_hbm.at[pl.ds(base, CH)], dst_v)
        pltpu.sync_copy(ones_v, acc.at[dst_v], add=True)
        return ()

    lax.fori_loop(0, N_FULL, chunk, ())

    tbase = base_w + N_FULL * CH
    pltpu.sync_copy(dst_hbm.at[pl.ds(tbase, TAIL)], dst_t)
    pltpu.sync_copy(ones_v.at[pl.ds(0, TAIL)], acc.at[dst_t], add=True)

    plsc.subcore_barrier()
    pltpu.sync_copy(acc.at[pl.ds(r0, ROWS_PER_TILE)],
                    out_hbm.at[cid, pl.ds(r0, ROWS_PER_TILE)])


@functools.partial(
    pl.kernel,
    out_type=jax.ShapeDtypeStruct((NC, N_PAD, H), jnp.float32),
    mesh=_mesh,
    scratch_types=[
        pltpu.VMEM((CH,), jnp.int32),
        pltpu.VMEM((CH,), jnp.int32),
        pltpu.VMEM((CH,), jnp.int32),
        pltpu.VMEM((CH,), jnp.int32),
        pltpu.VMEM((CH, H), jnp.float32),
        pltpu.VMEM((CH, H), jnp.float32),
        pltpu.VMEM((TAIL,), jnp.int32),
        pltpu.VMEM((TAIL,), jnp.int32),
        pltpu.VMEM((TAIL, H), jnp.float32),
        pltpu.VMEM_SHARED((N_PAD, H), jnp.float32),
        pltpu.SemaphoreType.DMA,
        pltpu.SemaphoreType.DMA,
        pltpu.SemaphoreType.DMA,
        pltpu.SemaphoreType.DMA,
    ],
)
def _agg_kernel(src_hbm, dst_hbm, y_hbm, out_hbm,
                src_a, src_b, dst_a, dst_b, rows_a, rows_b,
                src_t, dst_t, rows_t, acc,
                sem_ia, sem_ib, sem_ga, sem_gb):
    cid = lax.axis_index("c")
    sid = lax.axis_index("s")
    w = sid * NC + cid
    z16 = jnp.zeros((16,), jnp.float32)

    def zrow(i, _):
        for t in range(H // 16):
            rows_a[i, pl.ds(t * 16, 16)] = z16
        return ()

    lax.fori_loop(0, CH, zrow, ())
    r0 = sid * ROWS_PER_TILE
    for k in range(ROWS_PER_TILE // CH):
        pltpu.sync_copy(rows_a, acc.at[pl.ds(r0 + k * CH, CH)])
    plsc.subcore_barrier()

    base_w = w * E_PER_W

    pltpu.async_copy(src_hbm.at[pl.ds(base_w, CH)], src_a, sem_ia)
    pltpu.async_copy(dst_hbm.at[pl.ds(base_w, CH)], dst_a, sem_ia)
    pltpu.async_copy(src_hbm.at[pl.ds(base_w + CH, CH)], src_b, sem_ib)
    pltpu.async_copy(dst_hbm.at[pl.ds(base_w + CH, CH)], dst_b, sem_ib)
    pltpu.make_async_copy(src_hbm.at[pl.ds(base_w, CH)], src_a, sem_ia).wait()
    pltpu.async_copy(y_hbm.at[src_a], rows_a, sem_ga)
    pltpu.make_async_copy(src_hbm.at[pl.ds(base_w, CH)], src_b, sem_ib).wait()
    pltpu.async_copy(y_hbm.at[src_b], rows_b, sem_gb)

    NPAIR = N_FULL // 2

    def pair(i, _):
        j0 = 2 * i
        more = i < NPAIR - 1
        pltpu.make_async_copy(y_hbm.at[src_a], rows_a, sem_ga).wait()

        @pl.when(more)
        def _():
            pltpu.async_copy(src_hbm.at[pl.ds(base_w + (j0 + 2) * CH, CH)],
                             src_a, sem_ia)

        pltpu.make_async_copy(dst_hbm.at[pl.ds(base_w, CH)], dst_a,
                              sem_ia).wait()
        pltpu.sync_copy(rows_a, acc.at[dst_a], add=True)

        @pl.when(more)
        def _():
            pltpu.async_copy(dst_hbm.at[pl.ds(base_w + (j0 + 2) * CH, CH)],
                             dst_a, sem_ia)

        pltpu.make_async_copy(y_hbm.at[src_b], rows_b, sem_gb).wait()

        @pl.when(more)
        def _():
            pltpu.async_copy(src_hbm.at[pl.ds(base_w + (j0 + 3) * CH, CH)],
                             src_b, sem_ib)

        pltpu.make_async_copy(dst_hbm.at[pl.ds(base_w, CH)], dst_b,
                              sem_ib).wait()
        pltpu.sync_copy(rows_b, acc.at[dst_b], add=True)

        @pl.when(more)
        def _():
            pltpu.async_copy(dst_hbm.at[pl.ds(base_w + (j0 + 3) * CH, CH)],
                             dst_b, sem_ib)
            pltpu.make_async_copy(src_hbm.at[pl.ds(base_w, CH)], src_a,
                                  sem_ia).wait()
            pltpu.async_copy(y_hbm.at[src_a], rows_a, sem_ga)
            pltpu.make_async_copy(src_hbm.at[pl.ds(base_w, CH)], src_b,
                                  sem_ib).wait()
            pltpu.async_copy(y_hbm.at[src_b], rows_b, sem_gb)

        return ()

    lax.fori_loop(0, NPAIR, pair, ())

    tbase = base_w + N_FULL * CH
    pltpu.sync_copy(src_hbm.at[pl.ds(tbase, TAIL)], src_t)
    pltpu.sync_copy(dst_hbm.at[pl.ds(tbase, TAIL)], dst_t)
    pltpu.async_copy(y_hbm.at[src_t], rows_t, sem_ga).wait()
    pltpu.sync_copy(rows_t, acc.at[dst_t], add=True)

    plsc.subcore_barrier()
    pltpu.sync_copy(acc.at[pl.ds(r0, ROWS_PER_TILE)],
                    out_hbm.at[cid, pl.ds(r0, ROWS_PER_TILE)])


_BLK = 1000
_NBLK = N // _BLK


def _pre1_body(degp_ref, x_ref, w_ref, y_ref, dinv_ref):
    deg = jnp.sum(degp_ref[...], axis=1) * (1.0 / DW) + 1.0
    dinv = lax.rsqrt(deg)
    xw = jnp.dot(x_ref[...], w_ref[...], preferred_element_type=jnp.float32)
    y_ref[...] = dinv[:, None] * xw
    dinv_ref[...] = dinv[:, None]


_pre1 = pl.pallas_call(
    _pre1_body,
    grid=(_NBLK,),
    in_specs=[
        pl.BlockSpec((_BLK, NC * DW), lambda i: (i, 0)),
        pl.BlockSpec((_BLK, H), lambda i: (i, 0)),
        pl.BlockSpec((H, H), lambda i: (0, 0)),
    ],
    out_specs=(
        pl.BlockSpec((_BLK, H), lambda i: (i, 0)),
        pl.BlockSpec((_BLK, 1), lambda i: (i, 0)),
    ),
    out_shape=(
        jax.ShapeDtypeStruct((N, H), jnp.float32),
        jax.ShapeDtypeStruct((N, 1), jnp.float32),
    ),
)


def _trans_body(aggp_ref, y_ref, dinv_ref, b_ref, w_ref, out_ref):
    a = aggp_ref[0] + aggp_ref[1] + y_ref[...]
    h = jnp.tanh(dinv_ref[...] * a + b_ref[...])
    out_ref[...] = dinv_ref[...] * jnp.dot(
        h, w_ref[...], preferred_element_type=jnp.float32)


_trans = pl.pallas_call(
    _trans_body,
    grid=(_NBLK,),
    in_specs=[
        pl.BlockSpec((NC, _BLK, H), lambda i: (0, i, 0)),
        pl.BlockSpec((_BLK, H), lambda i: (i, 0)),
        pl.BlockSpec((_BLK, 1), lambda i: (i, 0)),
        pl.BlockSpec((1, H), lambda i: (0, 0)),
        pl.BlockSpec((H, H), lambda i: (0, 0)),
    ],
    out_specs=pl.BlockSpec((_BLK, H), lambda i: (i, 0)),
    out_shape=jax.ShapeDtypeStruct((N, H), jnp.float32),
)


def _final_body(aggp_ref, y_ref, dinv_ref, b_ref, batch_ref, wl_ref, bl_ref,
                out_ref, seg_ref, cnt_ref):
    i = pl.program_id(0)

    @pl.when(i == 0)
    def _():
        seg_ref[...] = jnp.zeros_like(seg_ref)
        cnt_ref[...] = jnp.zeros_like(cnt_ref)

    a = aggp_ref[0] + aggp_ref[1] + y_ref[...]
    h = dinv_ref[...] * a + b_ref[...]
    bt = batch_ref[...].reshape(1, _BLK)
    oh = (lax.broadcasted_iota(jnp.int32, (G, _BLK), 0)
          == jnp.broadcast_to(bt, (G, _BLK))).astype(jnp.float32)
    seg_ref[...] += jnp.dot(oh, h, preferred_element_type=jnp.float32)
    cnt_ref[...] += jnp.sum(oh, axis=1, keepdims=True)

    @pl.when(i == _NBLK - 1)
    def _():
        cnt = cnt_ref[...]
        mean = jnp.where(cnt > 0, seg_ref[...] / jnp.maximum(cnt, 1.0), 0.0)
        out_ref[...] = jnp.dot(
            mean, wl_ref[...], preferred_element_type=jnp.float32) + bl_ref[...]


_final = pl.pallas_call(
    _final_body,
    grid=(_NBLK,),
    in_specs=[
        pl.BlockSpec((NC, _BLK, H), lambda i: (0, i, 0)),
        pl.BlockSpec((_BLK, H), lambda i: (i, 0)),
        pl.BlockSpec((_BLK, 1), lambda i: (i, 0)),
        pl.BlockSpec((1, H), lambda i: (0, 0)),
        pl.BlockSpec((1, 1, _BLK), lambda i: (i, 0, 0)),
        pl.BlockSpec((H, 1), lambda i: (0, 0)),
        pl.BlockSpec((1, 1), lambda i: (0, 0)),
    ],
    out_specs=pl.BlockSpec((G, 1), lambda i: (0, 0)),
    out_shape=jax.ShapeDtypeStruct((G, 1), jnp.float32),
    scratch_shapes=[
        pltpu.VMEM((G, H), jnp.float32),
        pltpu.VMEM((G, 1), jnp.float32),
    ],
)


def kernel(x, edge_index, batch, W1, b1, W2, b2, W3, b3, Wl, bl):
    src = edge_index[0]
    dst = edge_index[1]
    deg_p = _deg_kernel(dst)
    y1, dinv = _pre1(deg_p.transpose(1, 0, 2).reshape(N_PAD, NC * DW), x, W1)
    agg1 = _agg_kernel(src, dst, y1)
    y2 = _trans(agg1, y1, dinv, b1.reshape(1, H), W2)
    agg2 = _agg_kernel(src, dst, y2)
    y3 = _trans(agg2, y2, dinv, b2.reshape(1, H), W3)
    agg3 = _agg_kernel(src, dst, y3)
    out = _final(agg3, y3, dinv, b3.reshape(1, H),
                 batch.reshape(_NBLK, 1, _BLK), Wl, bl.reshape(1, 1))
    return out
